# baseline (speedup 1.0000x reference)
import functools

import jax
import jax.numpy as jnp
from jax import lax
from jax.experimental import pallas as pl
from jax.experimental.pallas import tpu as pltpu
from jax.experimental.pallas import tpu_sc as plsc

N_NODES = 10000
N_EDGES = 320000
IN_DIM = 128
HID = 16
HEADS = 8
OUT_DIM = 64

CHUNK = 128
NTILES = 32
NPAD = 10240
ROWS_PER_TILE = NPAD // 16
NTAB = N_NODES + 16

E_TOT = N_EDGES + N_NODES
E_PAD = ((E_TOT + 2 * CHUNK * 16 - 1) // (2 * CHUNK * 16)) * (2 * CHUNK * 16)
EDGES_PER_TILE = E_PAD // 16
NCHUNKS = EDGES_PER_TILE // CHUNK


def _sc_edge_layer(C):
  CH = C // 2
  nvec = CH // 16
  mesh = plsc.VectorSubcoreMesh(core_axis_name="c", subcore_axis_name="s")

  @functools.partial(
      pl.kernel,
      out_type=[
          jax.ShapeDtypeStruct((2, NPAD, CH), jnp.float32),
          jax.ShapeDtypeStruct((2, NPAD, 16), jnp.float32),
      ],
      mesh=mesh,
      compiler_params=pltpu.CompilerParams(use_tc_tiling_on_sc=False),
      scratch_types=[
          pltpu.VMEM((NCHUNKS, CHUNK), jnp.int32),
          pltpu.VMEM((NCHUNKS, CHUNK), jnp.int32),
          pltpu.VMEM((CHUNK,), jnp.int32),
          pltpu.VMEM((CHUNK,), jnp.int32),
          pltpu.VMEM((CHUNK,), jnp.int32),
          pltpu.VMEM((CHUNK,), jnp.int32),
          pltpu.VMEM((CHUNK,), jnp.int32),
          pltpu.VMEM((CHUNK,), jnp.int32),
          pltpu.VMEM((CHUNK, 16), jnp.float32),
          pltpu.VMEM((CHUNK, 16), jnp.float32),
          pltpu.VMEM((CHUNK, 16), jnp.float32),
          pltpu.VMEM((CHUNK, 16), jnp.float32),
          pltpu.VMEM((CHUNK, 16), jnp.float32),
          pltpu.VMEM((CHUNK, 16), jnp.float32),
          pltpu.VMEM((CHUNK, CH), jnp.float32),
          pltpu.VMEM((CHUNK, CH), jnp.float32),
          pltpu.VMEM((2, 2, 16), jnp.float32),
          pltpu.VMEM_SHARED((NPAD, CH), jnp.float32),
          pltpu.VMEM_SHARED((NPAD, 16), jnp.float32),
          pltpu.SemaphoreType.DMA,
          pltpu.SemaphoreType.DMA,
          pltpu.SemaphoreType.DMA,
          pltpu.SemaphoreType.DMA,
          pltpu.SemaphoreType.DMA,
          pltpu.SemaphoreType.DMA,
      ],
  )
  def k(h_hbm, asrc_hbm, adst_hbm, m_hbm, src2_hbm, dst2_hbm, zc_hbm, z16_hbm,
        acc_out, den_out,
        six, dix, st0, st1, dt0, dt1, dr0, dr1, asr0, asr1, adr0, adr1,
        wb0, wb1, hr0, hr1, mbuf, acc_sh, den_sh,
        sa0, sa1, sb0, sb1, sh0, sh1):
    cid = lax.axis_index("c")
    sid = lax.axis_index("s")
    row0 = sid * ROWS_PER_TILE

    pltpu.sync_copy(zc_hbm.at[pl.ds(row0, ROWS_PER_TILE)],
                    acc_sh.at[pl.ds(row0, ROWS_PER_TILE)])
    pltpu.sync_copy(z16_hbm.at[pl.ds(row0, ROWS_PER_TILE)],
                    den_sh.at[pl.ds(row0, ROWS_PER_TILE)])
    pltpu.sync_copy(m_hbm, mbuf)
    pltpu.sync_copy(src2_hbm.at[pl.ds(sid * NCHUNKS, NCHUNKS)], six)
    pltpu.sync_copy(dst2_hbm.at[pl.ds(sid * NCHUNKS, NCHUNKS)], dix)
    m0 = mbuf[0, 0, :] + mbuf[0, 1, :]
    m1 = mbuf[1, 0, :] + mbuf[1, 1, :]
    mvec = jnp.where(cid == 0, m0, m1)
    off = cid * NTAB
    plsc.subcore_barrier()

    bufs = ((st0, dt0, dr0, asr0, adr0, wb0, hr0, sa0, sb0, sh0),
            (st1, dt1, dr1, asr1, adr1, wb1, hr1, sa1, sb1, sh1))

    def issue(i, b):
      st, dt, dr, asr, adr, _, hrows, sa, sb, sh = bufs[b]
      for j in range(CHUNK // 16):
        sl = pl.ds(j * 16, 16)
        d = dix[i, sl]
        st[sl] = six[i, sl] + off
        dt[sl] = d + off
        dr[sl] = d
      pltpu.async_copy(asrc_hbm.at[st], asr, sa)
      pltpu.async_copy(adst_hbm.at[dt], adr, sb)
      pltpu.async_copy(h_hbm.at[st], hrows, sh)

    def compute(i, b):
      st, dt, dr, asr, adr, wbuf, hrows, sa, sb, sh = bufs[b]
      pltpu.make_async_copy(asrc_hbm.at[st], asr, sa).wait()
      pltpu.make_async_copy(adst_hbm.at[dt], adr, sb).wait()

      def _wfun(e, c2):
        v = asr[e, :] + adr[e, :]
        v = jnp.where(v > 0, v, 0.2 * v)
        wbuf[e, :] = jnp.exp(v - mvec)
        return c2

      lax.fori_loop(0, CHUNK, _wfun, 0, unroll=8)

      pltpu.sync_copy(wbuf, den_sh.at[dr], add=True)
      pltpu.make_async_copy(h_hbm.at[st], hrows, sh).wait()

      def _sfun(e, c2):
        wv = wbuf[e, :]
        for j in range(nvec):
          w = wv[j] if C == 128 else wv[0]
          hrows[e, pl.ds(j * 16, 16)] = hrows[e, pl.ds(j * 16, 16)] * w
        return c2

      lax.fori_loop(0, CHUNK, _sfun, 0, unroll=4)

      pltpu.sync_copy(hrows, acc_sh.at[dr], add=True)

    issue(0, 0)

    def pair(g, carry):
      i0 = 2 * g
      issue(i0 + 1, 1)
      compute(i0, 0)
      issue(i0 + 2, 0)
      compute(i0 + 1, 1)
      return carry

    lax.fori_loop(0, NCHUNKS // 2 - 1, pair, 0)
    issue(NCHUNKS - 1, 1)
    compute(NCHUNKS - 2, 0)
    compute(NCHUNKS - 1, 1)
    plsc.subcore_barrier()
    pltpu.sync_copy(acc_sh.at[pl.ds(row0, ROWS_PER_TILE)],
                    acc_out.at[cid, pl.ds(row0, ROWS_PER_TILE)])
    pltpu.sync_copy(den_sh.at[pl.ds(row0, ROWS_PER_TILE)],
                    den_out.at[cid, pl.ds(row0, ROWS_PER_TILE)])

  return k


_edge128 = _sc_edge_layer(128)
_edge64 = _sc_edge_layer(64)

BLK = 1000
GRID = N_NODES // BLK


def _tc_a_body(x_ref, w1_ref, a1s_ref, a1d_ref,
               h_ref, asrc_ref, adst_ref, m_ref):
  h = jnp.dot(x_ref[...], w1_ref[...], preferred_element_type=jnp.float32)
  h_ref[...] = h
  asrc = jnp.dot(h, a1s_ref[...], preferred_element_type=jnp.float32)
  adst = jnp.dot(h, a1d_ref[...], preferred_element_type=jnp.float32)
  asrc_ref[...] = asrc
  adst_ref[...] = adst
  cur = jnp.concatenate([jnp.max(asrc, axis=0, keepdims=True),
                         jnp.max(adst, axis=0, keepdims=True)], axis=0)

  @pl.when(pl.program_id(0) == 0)
  def _():
    m_ref[...] = cur

  @pl.when(pl.program_id(0) != 0)
  def _():
    m_ref[...] = jnp.maximum(m_ref[...], cur)


def _tc_b_body(accL_ref, accR_ref, den_ref, e1_ref, b1_ref,
               w2_ref, a2s_ref, a2d_ref,
               h2_ref, asrc_ref, adst_ref, m_ref):
  den = den_ref[...] + 1e-16
  dexp = jnp.dot(den, e1_ref[...], preferred_element_type=jnp.float32)
  acc = jnp.concatenate([accL_ref[...], accR_ref[...]], axis=1)
  out1 = acc / dexp + b1_ref[...]
  out1 = jnp.where(out1 > 0, out1, jnp.exp(jnp.minimum(out1, 0.0)) - 1.0)
  h2 = jnp.dot(out1, w2_ref[...], preferred_element_type=jnp.float32)
  h2_ref[...] = h2
  asrc = jnp.dot(h2, a2s_ref[...], preferred_element_type=jnp.float32)
  adst = jnp.dot(h2, a2d_ref[...], preferred_element_type=jnp.float32)
  asrc_ref[...] = asrc
  adst_ref[...] = adst
  cur = jnp.concatenate([jnp.max(asrc, axis=0, keepdims=True),
                         jnp.max(adst, axis=0, keepdims=True)], axis=0)

  @pl.when(pl.program_id(0) == 0)
  def _():
    m_ref[...] = cur

  @pl.when(pl.program_id(0) != 0)
  def _():
    m_ref[...] = jnp.maximum(m_ref[...], cur)


def _tc_c_body(accL_ref, accR_ref, den_ref, e2_ref, b2_ref, out_ref):
  den = den_ref[...] + 1e-16
  dexp = jnp.dot(den, e2_ref[...], preferred_element_type=jnp.float32)
  acc = jnp.concatenate([accL_ref[...], accR_ref[...]], axis=1)
  out_ref[...] = acc / dexp + b2_ref[...]


def _full_spec(shape):
  return pl.BlockSpec(shape, lambda i: (0,) * len(shape))


def _row_spec(cols):
  return pl.BlockSpec((BLK, cols), lambda i: (i, 0))


def _blockdiag(att):
  H, Cc = att.shape
  eye = jnp.eye(16, dtype=att.dtype)[:H]
  return (att[:, :, None] * eye[:, None, :]).reshape(H * Cc, 16)


def _edge_tables(h, asrc, adst, m, roll):
  C = h.shape[1]
  pad16 = ((0, 16), (0, 0))
  hp = jnp.pad(h, pad16)
  ap = jnp.pad(asrc, pad16)
  bp = jnp.pad(adst, pad16)
  hstack = jnp.concatenate([hp[:, :C // 2], hp[:, C // 2:]], axis=0)
  a2 = jnp.concatenate([ap, jnp.roll(ap, -roll, axis=1)], axis=0)
  b2 = jnp.concatenate([bp, jnp.roll(bp, -roll, axis=1)], axis=0)
  m2 = jnp.stack([m, jnp.roll(m, -roll, axis=1)], axis=0)
  return hstack, a2, b2, m2


def kernel(x, edge_index, W1, att_src1, att_dst1, b1,
           W2, att_src2, att_dst2, b2):
  f32 = jnp.float32
  ar = jnp.arange(N_NODES, dtype=jnp.int32)
  padv = jnp.full((E_PAD - E_TOT,), N_NODES, dtype=jnp.int32)
  src = jnp.concatenate([edge_index[0].astype(jnp.int32), ar, padv])
  dst = jnp.concatenate([edge_index[1].astype(jnp.int32), ar, padv])
  src = src.reshape(16 * NCHUNKS, CHUNK)
  dst = dst.reshape(16 * NCHUNKS, CHUNK)

  A1s = _blockdiag(att_src1)
  A1d = _blockdiag(att_dst1)
  A2s = _blockdiag(att_src2)
  A2d = _blockdiag(att_dst2)
  E1 = jnp.concatenate([jnp.kron(jnp.eye(8, dtype=f32), jnp.ones((1, 16), f32)),
                        jnp.zeros((8, 128), f32)], axis=0)
  E2 = jnp.concatenate([jnp.ones((1, 64), f32),
                        jnp.zeros((15, 64), f32)], axis=0)
  z64 = jnp.zeros((NPAD, 64), f32)
  z32 = jnp.zeros((NPAD, 32), f32)
  z16 = jnp.zeros((NPAD, 16), f32)

  h1, asrc1, adst1, m1 = pl.pallas_call(
      _tc_a_body,
      grid=(GRID,),
      in_specs=[_row_spec(128), _full_spec((128, 128)),
                _full_spec((128, 16)), _full_spec((128, 16))],
      out_specs=[_row_spec(128), _row_spec(16), _row_spec(16),
                 _full_spec((2, 16))],
      out_shape=[jax.ShapeDtypeStruct((N_NODES, 128), f32),
                 jax.ShapeDtypeStruct((N_NODES, 16), f32),
                 jax.ShapeDtypeStruct((N_NODES, 16), f32),
                 jax.ShapeDtypeStruct((2, 16), f32)],
  )(x, W1, A1s, A1d)

  hs1, as1, ad1, m1s = _edge_tables(h1, asrc1, adst1, m1, roll=4)
  acc1, den1 = _edge128(hs1, as1, ad1, m1s, src, dst, z64, z16)

  h2, asrc2, adst2, m2 = pl.pallas_call(
      _tc_b_body,
      grid=(GRID,),
      in_specs=[_row_spec(64), _row_spec(64), _row_spec(16),
                _full_spec((16, 128)), _full_spec((1, 128)),
                _full_spec((128, 64)), _full_spec((64, 16)),
                _full_spec((64, 16))],
      out_specs=[_row_spec(64), _row_spec(16), _row_spec(16),
                 _full_spec((2, 16))],
      out_shape=[jax.ShapeDtypeStruct((N_NODES, 64), f32),
                 jax.ShapeDtypeStruct((N_NODES, 16), f32),
                 jax.ShapeDtypeStruct((N_NODES, 16), f32),
                 jax.ShapeDtypeStruct((2, 16), f32)],
  )(acc1[0, :N_NODES], acc1[1, :N_NODES], den1[0, :N_NODES],
    E1, b1.reshape(1, 128), W2, A2s, A2d)

  hs2, as2, ad2, m2s = _edge_tables(h2, asrc2, adst2, m2, roll=0)
  acc2, den2 = _edge64(hs2, as2, ad2, m2s, src, dst, z32, z16)

  out = pl.pallas_call(
      _tc_c_body,
      grid=(GRID,),
      in_specs=[_row_spec(32), _row_spec(32), _row_spec(16),
                _full_spec((16, 64)), _full_spec((1, 64))],
      out_specs=_row_spec(64),
      out_shape=jax.ShapeDtypeStruct((N_NODES, 64), f32),
  )(acc2[0, :N_NODES], acc2[1, :N_NODES], den2[0, :N_NODES],
    E2, b2.reshape(1, 64))
  return out


# speedup vs baseline: 1.0205x; 1.0205x over previous
import functools

import jax
import jax.numpy as jnp
from jax import lax
from jax.experimental import pallas as pl
from jax.experimental.pallas import tpu as pltpu
from jax.experimental.pallas import tpu_sc as plsc

N_NODES = 10000
N_EDGES = 320000
IN_DIM = 128
HID = 16
HEADS = 8
OUT_DIM = 64

CHUNK = 128
NTILES = 32
NPAD = 10240
ROWS_PER_TILE = NPAD // 16
NTAB = N_NODES + 16

E_TOT = N_EDGES + N_NODES
E_PAD = ((E_TOT + 2 * CHUNK * 16 - 1) // (2 * CHUNK * 16)) * (2 * CHUNK * 16)
EDGES_PER_TILE = E_PAD // 16
NCHUNKS = EDGES_PER_TILE // CHUNK


def _sc_edge_layer(C):
  CH = C // 2
  nvec = CH // 16
  mesh = plsc.VectorSubcoreMesh(core_axis_name="c", subcore_axis_name="s")

  @functools.partial(
      pl.kernel,
      out_type=[
          jax.ShapeDtypeStruct((2, NPAD, CH), jnp.float32),
          jax.ShapeDtypeStruct((2, NPAD, 16), jnp.float32),
      ],
      mesh=mesh,
      compiler_params=pltpu.CompilerParams(use_tc_tiling_on_sc=False),
      scratch_types=[
          pltpu.VMEM((NCHUNKS, CHUNK), jnp.int32),
          pltpu.VMEM((NCHUNKS, CHUNK), jnp.int32),
          pltpu.VMEM((CHUNK,), jnp.int32),
          pltpu.VMEM((CHUNK,), jnp.int32),
          pltpu.VMEM((CHUNK,), jnp.int32),
          pltpu.VMEM((CHUNK,), jnp.int32),
          pltpu.VMEM((CHUNK,), jnp.int32),
          pltpu.VMEM((CHUNK,), jnp.int32),
          pltpu.VMEM((CHUNK, 16), jnp.float32),
          pltpu.VMEM((CHUNK, 16), jnp.float32),
          pltpu.VMEM((CHUNK, 16), jnp.float32),
          pltpu.VMEM((CHUNK, 16), jnp.float32),
          pltpu.VMEM((CHUNK, 16), jnp.float32),
          pltpu.VMEM((CHUNK, 16), jnp.float32),
          pltpu.VMEM((CHUNK, CH), jnp.float32),
          pltpu.VMEM((CHUNK, CH), jnp.float32),
          pltpu.VMEM((2, 2, 16), jnp.float32),
          pltpu.VMEM_SHARED((NPAD, CH), jnp.float32),
          pltpu.VMEM_SHARED((NPAD, 16), jnp.float32),
          pltpu.SemaphoreType.DMA,
          pltpu.SemaphoreType.DMA,
          pltpu.SemaphoreType.DMA,
          pltpu.SemaphoreType.DMA,
          pltpu.SemaphoreType.DMA,
          pltpu.SemaphoreType.DMA,
      ],
  )
  def k(h_hbm, asrc_hbm, adst_hbm, m_hbm, src2_hbm, dst2_hbm, zc_hbm, z16_hbm,
        acc_out, den_out,
        six, dix, st0, st1, dt0, dt1, dr0, dr1, asr0, asr1, adr0, adr1,
        wb0, wb1, hr0, hr1, mbuf, acc_sh, den_sh,
        sa0, sa1, sb0, sb1, sh0, sh1):
    cid = lax.axis_index("c")
    sid = lax.axis_index("s")
    row0 = sid * ROWS_PER_TILE

    pltpu.sync_copy(zc_hbm.at[pl.ds(row0, ROWS_PER_TILE)],
                    acc_sh.at[pl.ds(row0, ROWS_PER_TILE)])
    pltpu.sync_copy(z16_hbm.at[pl.ds(row0, ROWS_PER_TILE)],
                    den_sh.at[pl.ds(row0, ROWS_PER_TILE)])
    pltpu.sync_copy(m_hbm, mbuf)
    pltpu.sync_copy(src2_hbm.at[pl.ds(sid * NCHUNKS, NCHUNKS)], six)
    pltpu.sync_copy(dst2_hbm.at[pl.ds(sid * NCHUNKS, NCHUNKS)], dix)
    m0 = mbuf[0, 0, :] + mbuf[0, 1, :]
    m1 = mbuf[1, 0, :] + mbuf[1, 1, :]
    mvec = jnp.where(cid == 0, m0, m1)
    off = cid * NTAB
    plsc.subcore_barrier()

    bufs = ((st0, dt0, dr0, asr0, adr0, wb0, hr0, sa0, sb0, sh0),
            (st1, dt1, dr1, asr1, adr1, wb1, hr1, sa1, sb1, sh1))

    def issue(i, b):
      st, dt, dr, asr, adr, _, hrows, sa, sb, sh = bufs[b]
      for j in range(CHUNK // 16):
        sl = pl.ds(j * 16, 16)
        d = dix[i, sl]
        st[sl] = six[i, sl] + off
        dt[sl] = d + off
        dr[sl] = d
      pltpu.async_copy(h_hbm.at[st], hrows, sh)
      pltpu.async_copy(asrc_hbm.at[st], asr, sa)
      pltpu.async_copy(adst_hbm.at[dt], adr, sb)

    def compute(i, b):
      st, dt, dr, asr, adr, wbuf, hrows, sa, sb, sh = bufs[b]
      pltpu.make_async_copy(asrc_hbm.at[st], asr, sa).wait()
      pltpu.make_async_copy(adst_hbm.at[dt], adr, sb).wait()
      pltpu.make_async_copy(h_hbm.at[st], hrows, sh).wait()

      def _fun(e, c2):
        v = asr[e, :] + adr[e, :]
        v = jnp.where(v > 0, v, 0.2 * v)
        wv = jnp.exp(v - mvec)
        wbuf[e, :] = wv
        for j in range(nvec):
          w = wv[j] if C == 128 else wv[0]
          hrows[e, pl.ds(j * 16, 16)] = hrows[e, pl.ds(j * 16, 16)] * w
        return c2

      lax.fori_loop(0, CHUNK, _fun, 0, unroll=8)

      pltpu.sync_copy(wbuf, den_sh.at[dr], add=True)
      pltpu.sync_copy(hrows, acc_sh.at[dr], add=True)

    issue(0, 0)

    def pair(g, carry):
      i0 = 2 * g
      issue(i0 + 1, 1)
      compute(i0, 0)
      issue(i0 + 2, 0)
      compute(i0 + 1, 1)
      return carry

    lax.fori_loop(0, NCHUNKS // 2 - 1, pair, 0)
    issue(NCHUNKS - 1, 1)
    compute(NCHUNKS - 2, 0)
    compute(NCHUNKS - 1, 1)
    plsc.subcore_barrier()
    pltpu.sync_copy(acc_sh.at[pl.ds(row0, ROWS_PER_TILE)],
                    acc_out.at[cid, pl.ds(row0, ROWS_PER_TILE)])
    pltpu.sync_copy(den_sh.at[pl.ds(row0, ROWS_PER_TILE)],
                    den_out.at[cid, pl.ds(row0, ROWS_PER_TILE)])

  return k


_edge128 = _sc_edge_layer(128)
_edge64 = _sc_edge_layer(64)

BLK = 1000
GRID = N_NODES // BLK


def _tc_a_body(x_ref, w1_ref, a1s_ref, a1d_ref,
               h_ref, asrc_ref, adst_ref, m_ref):
  h = jnp.dot(x_ref[...], w1_ref[...], preferred_element_type=jnp.float32)
  h_ref[...] = h
  asrc = jnp.dot(h, a1s_ref[...], preferred_element_type=jnp.float32)
  adst = jnp.dot(h, a1d_ref[...], preferred_element_type=jnp.float32)
  asrc_ref[...] = asrc
  adst_ref[...] = adst
  cur = jnp.concatenate([jnp.max(asrc, axis=0, keepdims=True),
                         jnp.max(adst, axis=0, keepdims=True)], axis=0)

  @pl.when(pl.program_id(0) == 0)
  def _():
    m_ref[...] = cur

  @pl.when(pl.program_id(0) != 0)
  def _():
    m_ref[...] = jnp.maximum(m_ref[...], cur)


def _tc_b_body(accL_ref, accR_ref, den_ref, e1_ref, b1_ref,
               w2_ref, a2s_ref, a2d_ref,
               h2_ref, asrc_ref, adst_ref, m_ref):
  den = den_ref[...] + 1e-16
  dexp = jnp.dot(den, e1_ref[...], preferred_element_type=jnp.float32)
  acc = jnp.concatenate([accL_ref[...], accR_ref[...]], axis=1)
  out1 = acc / dexp + b1_ref[...]
  out1 = jnp.where(out1 > 0, out1, jnp.exp(jnp.minimum(out1, 0.0)) - 1.0)
  h2 = jnp.dot(out1, w2_ref[...], preferred_element_type=jnp.float32)
  h2_ref[...] = h2
  asrc = jnp.dot(h2, a2s_ref[...], preferred_element_type=jnp.float32)
  adst = jnp.dot(h2, a2d_ref[...], preferred_element_type=jnp.float32)
  asrc_ref[...] = asrc
  adst_ref[...] = adst
  cur = jnp.concatenate([jnp.max(asrc, axis=0, keepdims=True),
                         jnp.max(adst, axis=0, keepdims=True)], axis=0)

  @pl.when(pl.program_id(0) == 0)
  def _():
    m_ref[...] = cur

  @pl.when(pl.program_id(0) != 0)
  def _():
    m_ref[...] = jnp.maximum(m_ref[...], cur)


def _tc_c_body(accL_ref, accR_ref, den_ref, e2_ref, b2_ref, out_ref):
  den = den_ref[...] + 1e-16
  dexp = jnp.dot(den, e2_ref[...], preferred_element_type=jnp.float32)
  acc = jnp.concatenate([accL_ref[...], accR_ref[...]], axis=1)
  out_ref[...] = acc / dexp + b2_ref[...]


def _full_spec(shape):
  return pl.BlockSpec(shape, lambda i: (0,) * len(shape))


def _row_spec(cols):
  return pl.BlockSpec((BLK, cols), lambda i: (i, 0))


def _blockdiag(att):
  H, Cc = att.shape
  eye = jnp.eye(16, dtype=att.dtype)[:H]
  return (att[:, :, None] * eye[:, None, :]).reshape(H * Cc, 16)


def _edge_tables(h, asrc, adst, m, roll):
  C = h.shape[1]
  pad16 = ((0, 16), (0, 0))
  hp = jnp.pad(h, pad16)
  ap = jnp.pad(asrc, pad16)
  bp = jnp.pad(adst, pad16)
  hstack = jnp.concatenate([hp[:, :C // 2], hp[:, C // 2:]], axis=0)
  a2 = jnp.concatenate([ap, jnp.roll(ap, -roll, axis=1)], axis=0)
  b2 = jnp.concatenate([bp, jnp.roll(bp, -roll, axis=1)], axis=0)
  m2 = jnp.stack([m, jnp.roll(m, -roll, axis=1)], axis=0)
  return hstack, a2, b2, m2


def kernel(x, edge_index, W1, att_src1, att_dst1, b1,
           W2, att_src2, att_dst2, b2):
  f32 = jnp.float32
  ar = jnp.arange(N_NODES, dtype=jnp.int32)
  padv = jnp.full((E_PAD - E_TOT,), N_NODES, dtype=jnp.int32)
  src = jnp.concatenate([edge_index[0].astype(jnp.int32), ar, padv])
  dst = jnp.concatenate([edge_index[1].astype(jnp.int32), ar, padv])
  src = src.reshape(16 * NCHUNKS, CHUNK)
  dst = dst.reshape(16 * NCHUNKS, CHUNK)

  A1s = _blockdiag(att_src1)
  A1d = _blockdiag(att_dst1)
  A2s = _blockdiag(att_src2)
  A2d = _blockdiag(att_dst2)
  E1 = jnp.concatenate([jnp.kron(jnp.eye(8, dtype=f32), jnp.ones((1, 16), f32)),
                        jnp.zeros((8, 128), f32)], axis=0)
  E2 = jnp.concatenate([jnp.ones((1, 64), f32),
                        jnp.zeros((15, 64), f32)], axis=0)
  z64 = jnp.zeros((NPAD, 64), f32)
  z32 = jnp.zeros((NPAD, 32), f32)
  z16 = jnp.zeros((NPAD, 16), f32)

  h1, asrc1, adst1, m1 = pl.pallas_call(
      _tc_a_body,
      grid=(GRID,),
      in_specs=[_row_spec(128), _full_spec((128, 128)),
                _full_spec((128, 16)), _full_spec((128, 16))],
      out_specs=[_row_spec(128), _row_spec(16), _row_spec(16),
                 _full_spec((2, 16))],
      out_shape=[jax.ShapeDtypeStruct((N_NODES, 128), f32),
                 jax.ShapeDtypeStruct((N_NODES, 16), f32),
                 jax.ShapeDtypeStruct((N_NODES, 16), f32),
                 jax.ShapeDtypeStruct((2, 16), f32)],
  )(x, W1, A1s, A1d)

  hs1, as1, ad1, m1s = _edge_tables(h1, asrc1, adst1, m1, roll=4)
  acc1, den1 = _edge128(hs1, as1, ad1, m1s, src, dst, z64, z16)

  h2, asrc2, adst2, m2 = pl.pallas_call(
      _tc_b_body,
      grid=(GRID,),
      in_specs=[_row_spec(64), _row_spec(64), _row_spec(16),
                _full_spec((16, 128)), _full_spec((1, 128)),
                _full_spec((128, 64)), _full_spec((64, 16)),
                _full_spec((64, 16))],
      out_specs=[_row_spec(64), _row_spec(16), _row_spec(16),
                 _full_spec((2, 16))],
      out_shape=[jax.ShapeDtypeStruct((N_NODES, 64), f32),
                 jax.ShapeDtypeStruct((N_NODES, 16), f32),
                 jax.ShapeDtypeStruct((N_NODES, 16), f32),
                 jax.ShapeDtypeStruct((2, 16), f32)],
  )(acc1[0, :N_NODES], acc1[1, :N_NODES], den1[0, :N_NODES],
    E1, b1.reshape(1, 128), W2, A2s, A2d)

  hs2, as2, ad2, m2s = _edge_tables(h2, asrc2, adst2, m2, roll=0)
  acc2, den2 = _edge64(hs2, as2, ad2, m2s, src, dst, z32, z16)

  out = pl.pallas_call(
      _tc_c_body,
      grid=(GRID,),
      in_specs=[_row_spec(32), _row_spec(32), _row_spec(16),
                _full_spec((16, 64)), _full_spec((1, 64))],
      out_specs=_row_spec(64),
      out_shape=jax.ShapeDtypeStruct((N_NODES, 64), f32),
  )(acc2[0, :N_NODES], acc2[1, :N_NODES], den2[0, :N_NODES],
    E2, b2.reshape(1, 64))
  return out


# speedup vs baseline: 1.0392x; 1.0183x over previous
import functools

import jax
import jax.numpy as jnp
from jax import lax
from jax.experimental import pallas as pl
from jax.experimental.pallas import tpu as pltpu
from jax.experimental.pallas import tpu_sc as plsc

N_NODES = 10000
N_EDGES = 320000
IN_DIM = 128
HID = 16
HEADS = 8
OUT_DIM = 64

CHUNK = 128
NTILES = 32
NPAD = 10240
ROWS_PER_TILE = NPAD // 16
NTAB = N_NODES + 16

E_TOT = N_EDGES + N_NODES
E_PAD = ((E_TOT + 2 * CHUNK * 16 - 1) // (2 * CHUNK * 16)) * (2 * CHUNK * 16)
EDGES_PER_TILE = E_PAD // 16
NCHUNKS = EDGES_PER_TILE // CHUNK


def _sc_edge_layer(C):
  CH = C // 2
  nvec = CH // 16
  mesh = plsc.VectorSubcoreMesh(core_axis_name="c", subcore_axis_name="s")

  @functools.partial(
      pl.kernel,
      out_type=[
          jax.ShapeDtypeStruct((2, NPAD, CH), jnp.float32),
          jax.ShapeDtypeStruct((2, NPAD, 16), jnp.float32),
      ],
      mesh=mesh,
      compiler_params=pltpu.CompilerParams(use_tc_tiling_on_sc=False),
      scratch_types=[
          pltpu.VMEM((NCHUNKS, CHUNK), jnp.int32),
          pltpu.VMEM((NCHUNKS, CHUNK), jnp.int32),
          pltpu.VMEM((CHUNK,), jnp.int32),
          pltpu.VMEM((CHUNK,), jnp.int32),
          pltpu.VMEM((CHUNK,), jnp.int32),
          pltpu.VMEM((CHUNK,), jnp.int32),
          pltpu.VMEM((CHUNK,), jnp.int32),
          pltpu.VMEM((CHUNK,), jnp.int32),
          pltpu.VMEM((CHUNK, 16), jnp.float32),
          pltpu.VMEM((CHUNK, 16), jnp.float32),
          pltpu.VMEM((CHUNK, 16), jnp.float32),
          pltpu.VMEM((CHUNK, 16), jnp.float32),
          pltpu.VMEM((CHUNK, 16), jnp.float32),
          pltpu.VMEM((CHUNK, 16), jnp.float32),
          pltpu.VMEM((CHUNK, CH), jnp.float32),
          pltpu.VMEM((CHUNK, CH), jnp.float32),
          pltpu.VMEM((2, 2, 16), jnp.float32),
          pltpu.VMEM_SHARED((NPAD, CH), jnp.float32),
          pltpu.VMEM_SHARED((NPAD, 16), jnp.float32),
          pltpu.SemaphoreType.DMA,
          pltpu.SemaphoreType.DMA,
          pltpu.SemaphoreType.DMA,
          pltpu.SemaphoreType.DMA,
          pltpu.SemaphoreType.DMA,
          pltpu.SemaphoreType.DMA,
          pltpu.SemaphoreType.DMA,
          pltpu.SemaphoreType.DMA,
          pltpu.SemaphoreType.DMA,
          pltpu.SemaphoreType.DMA,
      ],
  )
  def k(h_hbm, asrc_hbm, adst_hbm, m_hbm, src2_hbm, dst2_hbm, zc_hbm, z16_hbm,
        acc_out, den_out,
        six, dix, st0, st1, dt0, dt1, dr0, dr1, asr0, asr1, adr0, adr1,
        wb0, wb1, hr0, hr1, mbuf, acc_sh, den_sh,
        sa0, sa1, sb0, sb1, sh0, sh1, sw0, sw1, sc0, sc1):
    cid = lax.axis_index("c")
    sid = lax.axis_index("s")
    row0 = sid * ROWS_PER_TILE

    pltpu.sync_copy(zc_hbm.at[pl.ds(row0, ROWS_PER_TILE)],
                    acc_sh.at[pl.ds(row0, ROWS_PER_TILE)])
    pltpu.sync_copy(z16_hbm.at[pl.ds(row0, ROWS_PER_TILE)],
                    den_sh.at[pl.ds(row0, ROWS_PER_TILE)])
    pltpu.sync_copy(m_hbm, mbuf)
    pltpu.sync_copy(src2_hbm.at[pl.ds(sid * NCHUNKS, NCHUNKS)], six)
    pltpu.sync_copy(dst2_hbm.at[pl.ds(sid * NCHUNKS, NCHUNKS)], dix)
    m0 = mbuf[0, 0, :] + mbuf[0, 1, :]
    m1 = mbuf[1, 0, :] + mbuf[1, 1, :]
    mvec = jnp.where(cid == 0, m0, m1)
    off = cid * NTAB
    plsc.subcore_barrier()

    bufs = ((st0, dt0, dr0, asr0, adr0, wb0, hr0, sa0, sb0, sh0, sw0, sc0),
            (st1, dt1, dr1, asr1, adr1, wb1, hr1, sa1, sb1, sh1, sw1, sc1))

    def issue(i, b, first=False):
      st, dt, dr, asr, adr, wbuf, hrows, sa, sb, sh, sw, sc = bufs[b]
      if not first:
        pltpu.make_async_copy(wbuf, den_sh.at[dr], sw).wait()
        pltpu.make_async_copy(hrows, acc_sh.at[dr], sc).wait()
      for j in range(CHUNK // 16):
        sl = pl.ds(j * 16, 16)
        d = dix[i, sl]
        st[sl] = six[i, sl] + off
        dt[sl] = d + off
        dr[sl] = d
      pltpu.async_copy(h_hbm.at[st], hrows, sh)
      pltpu.async_copy(asrc_hbm.at[st], asr, sa)
      pltpu.async_copy(adst_hbm.at[dt], adr, sb)

    def compute(i, b):
      st, dt, dr, asr, adr, wbuf, hrows, sa, sb, sh, sw, sc = bufs[b]
      pltpu.make_async_copy(asrc_hbm.at[st], asr, sa).wait()
      pltpu.make_async_copy(adst_hbm.at[dt], adr, sb).wait()
      pltpu.make_async_copy(h_hbm.at[st], hrows, sh).wait()

      def _fun(e, c2):
        v = asr[e, :] + adr[e, :]
        v = jnp.where(v > 0, v, 0.2 * v)
        wv = jnp.exp(v - mvec)
        wbuf[e, :] = wv
        for j in range(nvec):
          w = wv[j] if C == 128 else wv[0]
          hrows[e, pl.ds(j * 16, 16)] = hrows[e, pl.ds(j * 16, 16)] * w
        return c2

      lax.fori_loop(0, CHUNK, _fun, 0, unroll=8)

      pltpu.async_copy(wbuf, den_sh.at[dr], sw, add=True)
      pltpu.async_copy(hrows, acc_sh.at[dr], sc, add=True)

    issue(0, 0, first=True)
    issue(1, 1, first=True)

    def pair(g, carry):
      i0 = 2 * g
      compute(i0, 0)
      issue(i0 + 2, 0)
      compute(i0 + 1, 1)
      issue(i0 + 3, 1)
      return carry

    lax.fori_loop(0, NCHUNKS // 2 - 1, pair, 0)
    compute(NCHUNKS - 2, 0)
    compute(NCHUNKS - 1, 1)
    for b in (0, 1):
      _, _, dr, _, _, wbuf, hrows, _, _, _, sw, sc = bufs[b]
      pltpu.make_async_copy(wbuf, den_sh.at[dr], sw).wait()
      pltpu.make_async_copy(hrows, acc_sh.at[dr], sc).wait()
    plsc.subcore_barrier()
    pltpu.sync_copy(acc_sh.at[pl.ds(row0, ROWS_PER_TILE)],
                    acc_out.at[cid, pl.ds(row0, ROWS_PER_TILE)])
    pltpu.sync_copy(den_sh.at[pl.ds(row0, ROWS_PER_TILE)],
                    den_out.at[cid, pl.ds(row0, ROWS_PER_TILE)])

  return k


_edge128 = _sc_edge_layer(128)
_edge64 = _sc_edge_layer(64)

BLK = 1000
GRID = N_NODES // BLK


def _tc_a_body(x_ref, w1_ref, a1s_ref, a1d_ref,
               h_ref, asrc_ref, adst_ref, m_ref):
  h = jnp.dot(x_ref[...], w1_ref[...], preferred_element_type=jnp.float32)
  h_ref[...] = h
  asrc = jnp.dot(h, a1s_ref[...], preferred_element_type=jnp.float32)
  adst = jnp.dot(h, a1d_ref[...], preferred_element_type=jnp.float32)
  asrc_ref[...] = asrc
  adst_ref[...] = adst
  cur = jnp.concatenate([jnp.max(asrc, axis=0, keepdims=True),
                         jnp.max(adst, axis=0, keepdims=True)], axis=0)

  @pl.when(pl.program_id(0) == 0)
  def _():
    m_ref[...] = cur

  @pl.when(pl.program_id(0) != 0)
  def _():
    m_ref[...] = jnp.maximum(m_ref[...], cur)


def _tc_b_body(accL_ref, accR_ref, den_ref, e1_ref, b1_ref,
               w2_ref, a2s_ref, a2d_ref,
               h2_ref, asrc_ref, adst_ref, m_ref):
  den = den_ref[...] + 1e-16
  dexp = jnp.dot(den, e1_ref[...], preferred_element_type=jnp.float32)
  acc = jnp.concatenate([accL_ref[...], accR_ref[...]], axis=1)
  out1 = acc / dexp + b1_ref[...]
  out1 = jnp.where(out1 > 0, out1, jnp.exp(jnp.minimum(out1, 0.0)) - 1.0)
  h2 = jnp.dot(out1, w2_ref[...], preferred_element_type=jnp.float32)
  h2_ref[...] = h2
  asrc = jnp.dot(h2, a2s_ref[...], preferred_element_type=jnp.float32)
  adst = jnp.dot(h2, a2d_ref[...], preferred_element_type=jnp.float32)
  asrc_ref[...] = asrc
  adst_ref[...] = adst
  cur = jnp.concatenate([jnp.max(asrc, axis=0, keepdims=True),
                         jnp.max(adst, axis=0, keepdims=True)], axis=0)

  @pl.when(pl.program_id(0) == 0)
  def _():
    m_ref[...] = cur

  @pl.when(pl.program_id(0) != 0)
  def _():
    m_ref[...] = jnp.maximum(m_ref[...], cur)


def _tc_c_body(accL_ref, accR_ref, den_ref, e2_ref, b2_ref, out_ref):
  den = den_ref[...] + 1e-16
  dexp = jnp.dot(den, e2_ref[...], preferred_element_type=jnp.float32)
  acc = jnp.concatenate([accL_ref[...], accR_ref[...]], axis=1)
  out_ref[...] = acc / dexp + b2_ref[...]


def _full_spec(shape):
  return pl.BlockSpec(shape, lambda i: (0,) * len(shape))


def _row_spec(cols):
  return pl.BlockSpec((BLK, cols), lambda i: (i, 0))


def _blockdiag(att):
  H, Cc = att.shape
  eye = jnp.eye(16, dtype=att.dtype)[:H]
  return (att[:, :, None] * eye[:, None, :]).reshape(H * Cc, 16)


def _edge_tables(h, asrc, adst, m, roll):
  C = h.shape[1]
  pad16 = ((0, 16), (0, 0))
  hp = jnp.pad(h, pad16)
  ap = jnp.pad(asrc, pad16)
  bp = jnp.pad(adst, pad16)
  hstack = jnp.concatenate([hp[:, :C // 2], hp[:, C // 2:]], axis=0)
  a2 = jnp.concatenate([ap, jnp.roll(ap, -roll, axis=1)], axis=0)
  b2 = jnp.concatenate([bp, jnp.roll(bp, -roll, axis=1)], axis=0)
  m2 = jnp.stack([m, jnp.roll(m, -roll, axis=1)], axis=0)
  return hstack, a2, b2, m2


def kernel(x, edge_index, W1, att_src1, att_dst1, b1,
           W2, att_src2, att_dst2, b2):
  f32 = jnp.float32
  ar = jnp.arange(N_NODES, dtype=jnp.int32)
  padv = jnp.full((E_PAD - E_TOT,), N_NODES, dtype=jnp.int32)
  src = jnp.concatenate([edge_index[0].astype(jnp.int32), ar, padv])
  dst = jnp.concatenate([edge_index[1].astype(jnp.int32), ar, padv])
  src = src.reshape(16 * NCHUNKS, CHUNK)
  dst = dst.reshape(16 * NCHUNKS, CHUNK)

  A1s = _blockdiag(att_src1)
  A1d = _blockdiag(att_dst1)
  A2s = _blockdiag(att_src2)
  A2d = _blockdiag(att_dst2)
  E1 = jnp.concatenate([jnp.kron(jnp.eye(8, dtype=f32), jnp.ones((1, 16), f32)),
                        jnp.zeros((8, 128), f32)], axis=0)
  E2 = jnp.concatenate([jnp.ones((1, 64), f32),
                        jnp.zeros((15, 64), f32)], axis=0)
  z64 = jnp.zeros((NPAD, 64), f32)
  z32 = jnp.zeros((NPAD, 32), f32)
  z16 = jnp.zeros((NPAD, 16), f32)

  h1, asrc1, adst1, m1 = pl.pallas_call(
      _tc_a_body,
      grid=(GRID,),
      in_specs=[_row_spec(128), _full_spec((128, 128)),
                _full_spec((128, 16)), _full_spec((128, 16))],
      out_specs=[_row_spec(128), _row_spec(16), _row_spec(16),
                 _full_spec((2, 16))],
      out_shape=[jax.ShapeDtypeStruct((N_NODES, 128), f32),
                 jax.ShapeDtypeStruct((N_NODES, 16), f32),
                 jax.ShapeDtypeStruct((N_NODES, 16), f32),
                 jax.ShapeDtypeStruct((2, 16), f32)],
  )(x, W1, A1s, A1d)

  hs1, as1, ad1, m1s = _edge_tables(h1, asrc1, adst1, m1, roll=4)
  acc1, den1 = _edge128(hs1, as1, ad1, m1s, src, dst, z64, z16)

  h2, asrc2, adst2, m2 = pl.pallas_call(
      _tc_b_body,
      grid=(GRID,),
      in_specs=[_row_spec(64), _row_spec(64), _row_spec(16),
                _full_spec((16, 128)), _full_spec((1, 128)),
                _full_spec((128, 64)), _full_spec((64, 16)),
                _full_spec((64, 16))],
      out_specs=[_row_spec(64), _row_spec(16), _row_spec(16),
                 _full_spec((2, 16))],
      out_shape=[jax.ShapeDtypeStruct((N_NODES, 64), f32),
                 jax.ShapeDtypeStruct((N_NODES, 16), f32),
                 jax.ShapeDtypeStruct((N_NODES, 16), f32),
                 jax.ShapeDtypeStruct((2, 16), f32)],
  )(acc1[0, :N_NODES], acc1[1, :N_NODES], den1[0, :N_NODES],
    E1, b1.reshape(1, 128), W2, A2s, A2d)

  hs2, as2, ad2, m2s = _edge_tables(h2, asrc2, adst2, m2, roll=0)
  acc2, den2 = _edge64(hs2, as2, ad2, m2s, src, dst, z32, z16)

  out = pl.pallas_call(
      _tc_c_body,
      grid=(GRID,),
      in_specs=[_row_spec(32), _row_spec(32), _row_spec(16),
                _full_spec((16, 64)), _full_spec((1, 64))],
      out_specs=_row_spec(64),
      out_shape=jax.ShapeDtypeStruct((N_NODES, 64), f32),
  )(acc2[0, :N_NODES], acc2[1, :N_NODES], den2[0, :N_NODES],
    E2, b2.reshape(1, 64))
  return out


# speedup vs baseline: 1.1751x; 1.1308x over previous
import functools

import jax
import jax.numpy as jnp
from jax import lax
from jax.experimental import pallas as pl
from jax.experimental.pallas import tpu as pltpu
from jax.experimental.pallas import tpu_sc as plsc

N_NODES = 10000
N_EDGES = 320000
IN_DIM = 128
HID = 16
HEADS = 8
OUT_DIM = 64

CHUNK = 112
NTILES = 32
NPAD = 10016
ROWS_PER_TILE = NPAD // 16
NTAB = N_NODES + 16

E_TOT = N_EDGES + N_NODES
E_PAD = ((E_TOT + 2 * CHUNK * NTILES - 1) // (2 * CHUNK * NTILES)) * (2 * CHUNK * NTILES)
EDGES_PER_TILE = E_PAD // NTILES
NCHUNKS = EDGES_PER_TILE // CHUNK


def _sc_edge_layer(C):
  W = C + 16
  nvec = C // 16
  mesh = plsc.VectorSubcoreMesh(core_axis_name="c", subcore_axis_name="s")

  @functools.partial(
      pl.kernel,
      out_type=jax.ShapeDtypeStruct((2, NPAD, W), jnp.float32),
      mesh=mesh,
      compiler_params=pltpu.CompilerParams(use_tc_tiling_on_sc=False),
      scratch_types=[
          pltpu.VMEM((CHUNK,), jnp.int32),
          pltpu.VMEM((CHUNK,), jnp.int32),
          pltpu.VMEM((CHUNK,), jnp.int32),
          pltpu.VMEM((CHUNK,), jnp.int32),
          pltpu.VMEM((CHUNK,), jnp.int32),
          pltpu.VMEM((CHUNK,), jnp.int32),
          pltpu.VMEM((CHUNK, 16), jnp.float32),
          pltpu.VMEM((CHUNK, 16), jnp.float32),
          pltpu.VMEM((CHUNK, W), jnp.float32),
          pltpu.VMEM((CHUNK, W), jnp.float32),
          pltpu.VMEM((2, 16), jnp.float32),
          pltpu.VMEM_SHARED((NPAD, W), jnp.float32),
          pltpu.SemaphoreType.DMA,
          pltpu.SemaphoreType.DMA,
          pltpu.SemaphoreType.DMA,
          pltpu.SemaphoreType.DMA,
          pltpu.SemaphoreType.DMA,
          pltpu.SemaphoreType.DMA,
          pltpu.SemaphoreType.DMA,
          pltpu.SemaphoreType.DMA,
          pltpu.SemaphoreType.DMA,
          pltpu.SemaphoreType.DMA,
      ],
  )
  def k(h_hbm, adst_hbm, m_hbm, src2_hbm, dst2_hbm, zc_hbm,
        acc_out,
        is0, is1, id0, id1, ds0, ds1, adr0, adr1, hr0, hr1, mbuf,
        acc_sh, si0, si1, sd0, sd1, sb0, sb1, sh0, sh1, sc0, sc1):
    cid = lax.axis_index("c")
    sid = lax.axis_index("s")
    row0 = sid * ROWS_PER_TILE

    pltpu.sync_copy(zc_hbm.at[pl.ds(row0, ROWS_PER_TILE)],
                    acc_sh.at[pl.ds(row0, ROWS_PER_TILE)])
    pltpu.sync_copy(m_hbm, mbuf)
    mvec = mbuf[0, :] + mbuf[1, :]
    wid = sid * 2 + cid
    crow0 = wid * NCHUNKS
    plsc.subcore_barrier()

    bufs = ((is0, id0, ds0, adr0, hr0, si0, sd0, sb0, sh0, sc0),
            (is1, id1, ds1, adr1, hr1, si1, sd1, sb1, sh1, sc1))

    def idx_issue(i, b):
      isb, idb, _, _, _, si, sd, _, _, _ = bufs[b]
      pltpu.async_copy(src2_hbm.at[crow0 + i], isb, si)
      pltpu.async_copy(dst2_hbm.at[crow0 + i], idb, sd)

    def idx_wait(i, b):
      isb, idb, _, _, _, si, sd, _, _, _ = bufs[b]
      pltpu.make_async_copy(src2_hbm.at[crow0 + i], isb, si).wait()
      pltpu.make_async_copy(dst2_hbm.at[crow0 + i], idb, sd).wait()

    def gather_issue(i, b, drain):
      isb, idb, dsb, adr, hrows, _, _, sb, sh, sc = bufs[b]
      if drain:
        pltpu.make_async_copy(hrows, acc_sh.at[dsb], sc).wait()
      pltpu.async_copy(h_hbm.at[isb], hrows, sh)
      pltpu.async_copy(adst_hbm.at[idb], adr, sb)

    def compute(i, b):
      isb, idb, dsb, adr, hrows, _, _, sb, sh, sc = bufs[b]
      pltpu.make_async_copy(adst_hbm.at[idb], adr, sb).wait()
      pltpu.make_async_copy(h_hbm.at[isb], hrows, sh).wait()

      def _fun(e, c2):
        v = hrows[e, pl.ds(C, 16)] + adr[e, :]
        v = jnp.maximum(v, 0.2 * v)
        wv = jnp.exp(v - mvec)
        hrows[e, pl.ds(C, 16)] = wv
        for j in range(nvec):
          w = wv[j] if C == 128 else wv[0]
          hrows[e, pl.ds(j * 16, 16)] = hrows[e, pl.ds(j * 16, 16)] * w
        return c2

      lax.fori_loop(0, CHUNK, _fun, 0, unroll=8)
      for j in range(CHUNK // 16):
        sl = pl.ds(j * 16, 16)
        dsb[sl] = idb[sl]
      pltpu.async_copy(hrows, acc_sh.at[dsb], sc, add=True)

    idx_issue(0, 0)
    idx_issue(1, 1)
    idx_wait(0, 0)
    gather_issue(0, 0, drain=False)
    idx_wait(1, 1)
    gather_issue(1, 1, drain=False)
    compute(0, 0)
    idx_issue(2, 0)
    idx_wait(2, 0)
    gather_issue(2, 0, drain=True)
    compute(1, 1)
    idx_issue(3, 1)

    def pair(g, carry):
      i0 = 2 * g
      idx_wait(i0 + 1, 1)
      gather_issue(i0 + 1, 1, drain=True)
      compute(i0, 0)
      idx_issue(i0 + 2, 0)
      idx_wait(i0 + 2, 0)
      gather_issue(i0 + 2, 0, drain=True)
      compute(i0 + 1, 1)
      idx_issue(i0 + 3, 1)
      return carry

    lax.fori_loop(1, NCHUNKS // 2 - 1, pair, 0)
    idx_wait(NCHUNKS - 1, 1)
    gather_issue(NCHUNKS - 1, 1, drain=True)
    compute(NCHUNKS - 2, 0)
    compute(NCHUNKS - 1, 1)
    for b in (0, 1):
      _, _, dsb, _, hrows, _, _, _, _, sc = bufs[b]
      pltpu.make_async_copy(hrows, acc_sh.at[dsb], sc).wait()
    plsc.subcore_barrier()
    pltpu.sync_copy(acc_sh.at[pl.ds(row0, ROWS_PER_TILE)],
                    acc_out.at[cid, pl.ds(row0, ROWS_PER_TILE)])

  return k


_edge128 = _sc_edge_layer(128)
_edge64 = _sc_edge_layer(64)

BLK = 1000
GRID = N_NODES // BLK


def _tc_a_body(x_ref, w1_ref, a1s_ref, a1d_ref,
               h_ref, adst_ref, m_ref):
  h = jnp.dot(x_ref[...], w1_ref[...], preferred_element_type=jnp.float32)
  asrc = jnp.dot(h, a1s_ref[...], preferred_element_type=jnp.float32)
  adst = jnp.dot(h, a1d_ref[...], preferred_element_type=jnp.float32)
  h_ref[...] = jnp.concatenate([h, asrc], axis=1)
  adst_ref[...] = adst
  cur = jnp.concatenate([jnp.max(asrc, axis=0, keepdims=True),
                         jnp.max(adst, axis=0, keepdims=True)], axis=0)

  @pl.when(pl.program_id(0) == 0)
  def _():
    m_ref[...] = cur

  @pl.when(pl.program_id(0) != 0)
  def _():
    m_ref[...] = jnp.maximum(m_ref[...], cur)


def _tc_b_body(accA_ref, accB_ref, e1_ref, b1_ref,
               w2_ref, a2s_ref, a2d_ref,
               h2_ref, adst_ref, m_ref):
  accw = accA_ref[...] + accB_ref[...]
  den = accw[:, 128:144] + 1e-16
  dexp = jnp.dot(den, e1_ref[...], preferred_element_type=jnp.float32)
  out1 = accw[:, :128] / dexp + b1_ref[...]
  out1 = jnp.where(out1 > 0, out1, jnp.exp(jnp.minimum(out1, 0.0)) - 1.0)
  h2 = jnp.dot(out1, w2_ref[...], preferred_element_type=jnp.float32)
  asrc = jnp.dot(h2, a2s_ref[...], preferred_element_type=jnp.float32)
  adst = jnp.dot(h2, a2d_ref[...], preferred_element_type=jnp.float32)
  h2_ref[...] = jnp.concatenate([h2, asrc], axis=1)
  adst_ref[...] = adst
  cur = jnp.concatenate([jnp.max(asrc, axis=0, keepdims=True),
                         jnp.max(adst, axis=0, keepdims=True)], axis=0)

  @pl.when(pl.program_id(0) == 0)
  def _():
    m_ref[...] = cur

  @pl.when(pl.program_id(0) != 0)
  def _():
    m_ref[...] = jnp.maximum(m_ref[...], cur)


def _tc_c_body(accA_ref, accB_ref, e2_ref, b2_ref, out_ref):
  accw = accA_ref[...] + accB_ref[...]
  den = accw[:, 64:80] + 1e-16
  dexp = jnp.dot(den, e2_ref[...], preferred_element_type=jnp.float32)
  out_ref[...] = accw[:, :64] / dexp + b2_ref[...]


def _full_spec(shape):
  return pl.BlockSpec(shape, lambda i: (0,) * len(shape))


def _row_spec(cols):
  return pl.BlockSpec((BLK, cols), lambda i: (i, 0))


def _blockdiag(att):
  H, Cc = att.shape
  eye = jnp.eye(16, dtype=att.dtype)[:H]
  return (att[:, :, None] * eye[:, None, :]).reshape(H * Cc, 16)


def kernel(x, edge_index, W1, att_src1, att_dst1, b1,
           W2, att_src2, att_dst2, b2):
  f32 = jnp.float32
  ar = jnp.arange(N_NODES, dtype=jnp.int32)
  padv = jnp.full((E_PAD - E_TOT,), N_NODES, dtype=jnp.int32)
  src = jnp.concatenate([edge_index[0].astype(jnp.int32), ar, padv])
  dst = jnp.concatenate([edge_index[1].astype(jnp.int32), ar, padv])
  src = src.reshape(NTILES * NCHUNKS, CHUNK)
  dst = dst.reshape(NTILES * NCHUNKS, CHUNK)

  A1s = _blockdiag(att_src1)
  A1d = _blockdiag(att_dst1)
  A2s = _blockdiag(att_src2)
  A2d = _blockdiag(att_dst2)
  E1 = jnp.concatenate([jnp.kron(jnp.eye(8, dtype=f32), jnp.ones((1, 16), f32)),
                        jnp.zeros((8, 128), f32)], axis=0)
  E2 = jnp.concatenate([jnp.ones((1, 64), f32),
                        jnp.zeros((15, 64), f32)], axis=0)
  z144 = jnp.zeros((NPAD, 144), f32)
  z80 = jnp.zeros((NPAD, 80), f32)
  pad16 = ((0, 16), (0, 0))

  ht1, adst1, m1 = pl.pallas_call(
      _tc_a_body,
      grid=(GRID,),
      in_specs=[_row_spec(128), _full_spec((128, 128)),
                _full_spec((128, 16)), _full_spec((128, 16))],
      out_specs=[_row_spec(144), _row_spec(16), _full_spec((2, 16))],
      out_shape=[jax.ShapeDtypeStruct((N_NODES, 144), f32),
                 jax.ShapeDtypeStruct((N_NODES, 16), f32),
                 jax.ShapeDtypeStruct((2, 16), f32)],
  )(x, W1, A1s, A1d)

  acc1 = _edge128(jnp.pad(ht1, pad16), jnp.pad(adst1, pad16), m1,
                  src, dst, z144)

  ht2, adst2, m2 = pl.pallas_call(
      _tc_b_body,
      grid=(GRID,),
      in_specs=[_row_spec(144), _row_spec(144),
                _full_spec((16, 128)), _full_spec((1, 128)),
                _full_spec((128, 64)), _full_spec((64, 16)),
                _full_spec((64, 16))],
      out_specs=[_row_spec(80), _row_spec(16), _full_spec((2, 16))],
      out_shape=[jax.ShapeDtypeStruct((N_NODES, 80), f32),
                 jax.ShapeDtypeStruct((N_NODES, 16), f32),
                 jax.ShapeDtypeStruct((2, 16), f32)],
  )(acc1[0, :N_NODES], acc1[1, :N_NODES],
    E1, b1.reshape(1, 128), W2, A2s, A2d)

  acc2 = _edge64(jnp.pad(ht2, pad16), jnp.pad(adst2, pad16), m2,
                 src, dst, z80)

  out = pl.pallas_call(
      _tc_c_body,
      grid=(GRID,),
      in_specs=[_row_spec(80), _row_spec(80),
                _full_spec((16, 64)), _full_spec((1, 64))],
      out_specs=_row_spec(64),
      out_shape=jax.ShapeDtypeStruct((N_NODES, 64), f32),
  )(acc2[0, :N_NODES], acc2[1, :N_NODES], E2, b2.reshape(1, 64))
  return out
